# baseline (speedup 1.0000x reference)
import functools

import jax
import jax.numpy as jnp
from jax.experimental import pallas as pl
from jax.experimental.pallas import tpu as pltpu

_IB = 4


def _mlp_mask_kernel(x_ref, m_ref, w1_ref, b1_ref, w2_ref, b2_ref, out_ref):
    bi, s, nj = x_ref.shape[1], x_ref.shape[2], x_ref.shape[3]
    heads = out_ref.shape[2]
    w1 = w1_ref[...]
    w2 = w2_ref[...]
    b1 = b1_ref[...]
    b2 = b2_ref[...]
    for g in range(bi // _IB):
        x = x_ref[0, g * _IB:(g + 1) * _IB].reshape(_IB * s, nj)
        h = (jnp.dot(w1, x.astype(jnp.bfloat16),
                     preferred_element_type=jnp.float32) + b1).astype(jnp.bfloat16)
        h = jnp.maximum(h, jnp.bfloat16(0.0))
        o = jnp.dot(w2, h, preferred_element_type=jnp.float32) + b2
        m = m_ref[0, g].astype(jnp.float32)
        og = o.reshape(_IB, heads, nj) * m.reshape(_IB, 1, nj)
        out_ref[0, g * _IB:(g + 1) * _IB] = og


@functools.partial(jax.jit, static_argnames=("block_i",))
def _run(xT, mask4, W1bd, b1bd, W2bd, b2bd, block_i=64):
    b, n, s, nj = xT.shape
    heads = W2bd.shape[0] // _IB
    grid = (b, n // block_i)
    return pl.pallas_call(
        _mlp_mask_kernel,
        grid=grid,
        in_specs=[
            pl.BlockSpec((1, block_i, s, nj), lambda ib, ii: (ib, ii, 0, 0)),
            pl.BlockSpec((1, block_i // _IB, _IB, nj), lambda ib, ii: (ib, ii, 0, 0)),
            pl.BlockSpec(W1bd.shape, lambda ib, ii: (0, 0)),
            pl.BlockSpec(b1bd.shape, lambda ib, ii: (0, 0)),
            pl.BlockSpec(W2bd.shape, lambda ib, ii: (0, 0)),
            pl.BlockSpec(b2bd.shape, lambda ib, ii: (0, 0)),
        ],
        out_specs=pl.BlockSpec((1, block_i, heads, nj), lambda ib, ii: (ib, ii, 0, 0)),
        out_shape=jax.ShapeDtypeStruct((b, n, heads, nj), jnp.float32),
        compiler_params=pltpu.CompilerParams(
            dimension_semantics=("parallel", "parallel"),
        ),
    )(xT, mask4, W1bd, b1bd, W2bd, b2bd)


def kernel(stacks, mask, W1, b1, W2, b2):
    b, n, _, _ = stacks.shape
    xT = jnp.swapaxes(stacks, 2, 3)
    mask4 = mask.reshape(b, n // _IB, _IB, n)
    eye = jnp.eye(_IB, dtype=jnp.float32)
    W1bd = jnp.kron(eye, W1.T).astype(jnp.bfloat16)
    W2bd = jnp.kron(eye, W2.T).astype(jnp.bfloat16)
    b1bd = jnp.tile(b1, _IB).reshape(-1, 1)
    b2bd = jnp.tile(b2, _IB).reshape(-1, 1)
    outT = _run(xT, mask4, W1bd, b1bd, W2bd, b2bd)
    return jnp.swapaxes(outT, 2, 3)


# speedup vs baseline: 1.0080x; 1.0080x over previous
import functools

import jax
import jax.numpy as jnp
from jax.experimental import pallas as pl
from jax.experimental.pallas import tpu as pltpu

_IB = 4


def _mlp_mask_kernel(x_ref, m_ref, w1_ref, b1_ref, w2_ref, b2_ref, out_ref):
    bi, s, nj = x_ref.shape[1], x_ref.shape[2], x_ref.shape[3]
    heads = out_ref.shape[2]
    w1 = w1_ref[...]
    w2 = w2_ref[...]
    b1 = b1_ref[...]
    b2 = b2_ref[...]
    for g in range(bi // _IB):
        x = x_ref[0, g * _IB:(g + 1) * _IB].reshape(_IB * s, nj)
        h = (jnp.dot(w1, x.astype(jnp.bfloat16),
                     preferred_element_type=jnp.float32) + b1).astype(jnp.bfloat16)
        h = jnp.maximum(h, jnp.bfloat16(0.0))
        o = jnp.dot(w2, h, preferred_element_type=jnp.float32) + b2
        m = m_ref[0, g].astype(jnp.float32)
        og = o.reshape(_IB, heads, nj) * m.reshape(_IB, 1, nj)
        out_ref[0, g * _IB:(g + 1) * _IB] = og


@functools.partial(jax.jit, static_argnames=("block_i",))
def _run(xT, mask4, W1bd, b1bd, W2bd, b2bd, block_i=128):
    b, n, s, nj = xT.shape
    heads = W2bd.shape[0] // _IB
    grid = (b, n // block_i)
    return pl.pallas_call(
        _mlp_mask_kernel,
        grid=grid,
        in_specs=[
            pl.BlockSpec((1, block_i, s, nj), lambda ib, ii: (ib, ii, 0, 0)),
            pl.BlockSpec((1, block_i // _IB, _IB, nj), lambda ib, ii: (ib, ii, 0, 0)),
            pl.BlockSpec(W1bd.shape, lambda ib, ii: (0, 0)),
            pl.BlockSpec(b1bd.shape, lambda ib, ii: (0, 0)),
            pl.BlockSpec(W2bd.shape, lambda ib, ii: (0, 0)),
            pl.BlockSpec(b2bd.shape, lambda ib, ii: (0, 0)),
        ],
        out_specs=pl.BlockSpec((1, block_i, heads, nj), lambda ib, ii: (ib, ii, 0, 0)),
        out_shape=jax.ShapeDtypeStruct((b, n, heads, nj), jnp.float32),
        compiler_params=pltpu.CompilerParams(
            dimension_semantics=("parallel", "parallel"),
        ),
    )(xT, mask4, W1bd, b1bd, W2bd, b2bd)


def kernel(stacks, mask, W1, b1, W2, b2):
    b, n, _, _ = stacks.shape
    xT = jnp.swapaxes(stacks, 2, 3)
    mask4 = mask.reshape(b, n // _IB, _IB, n)
    eye = jnp.eye(_IB, dtype=jnp.float32)
    W1bd = jnp.kron(eye, W1.T).astype(jnp.bfloat16)
    W2bd = jnp.kron(eye, W2.T).astype(jnp.bfloat16)
    b1bd = jnp.tile(b1, _IB).reshape(-1, 1)
    b2bd = jnp.tile(b2, _IB).reshape(-1, 1)
    outT = _run(xT, mask4, W1bd, b1bd, W2bd, b2bd)
    return jnp.swapaxes(outT, 2, 3)
